# baseline (speedup 1.0000x reference)
import jax
import jax.numpy as jnp
from jax import lax
from jax.experimental import pallas as pl
from jax.experimental.pallas import tpu as pltpu
from jax.experimental.pallas import tpu_sc as plsc

F = 26
V = 100000
D = 16
B = 16384

NC = 2
NS = 16
NW = NC * NS
L = 16

UNITS = F * D
UPW = UNITS // NW
QB = B // 4

_MESH = plsc.VectorSubcoreMesh(
    core_axis_name="c", subcore_axis_name="s", num_cores=NC, num_subcores=NS
)


def _body(xt_hbm, tab_hbm, out_hbm, row_v, xcol_v, ob0_v, ob1_v,
          sem_row, sem_row2, sem_w0, sem_w1):
    wid = lax.axis_index("s") * NC + lax.axis_index("c")
    obufs = (ob0_v, ob1_v)
    wsems = (sem_w0, sem_w1)
    pending = [None, None]

    for i in range(UPW):
        u = wid * UPW + i
        f = u // D
        d = u % D

        rowcp = pltpu.async_copy(tab_hbm.at[f, d], row_v, sem_row)
        if i == 0:
            pltpu.sync_copy(xt_hbm.at[f], xcol_v)
        else:
            @pl.when(d == 0)
            def _():
                pltpu.sync_copy(xt_hbm.at[f], xcol_v)
        rowcp.wait()

        for q in range(4):
            s = q % 2
            if pending[s] is not None:
                pending[s].wait()

            @plsc.parallel_loop(0, QB, step=L, unroll=16)
            def gather(k):
                idx = xcol_v[pl.ds(q * QB + k, L)]
                obufs[s][pl.ds(k, L)] = plsc.load_gather(row_v, [idx])

            pending[s] = pltpu.async_copy(
                obufs[s], out_hbm.at[f, d, pl.ds(q * QB, QB)], wsems[s]
            )

    pending[0].wait()
    pending[1].wait()


@jax.jit
def kernel(x, tables):
    xt = x.astype(jnp.int32).T
    tab_t = jnp.swapaxes(tables, 1, 2)
    run = pl.kernel(
        _body,
        out_type=jax.ShapeDtypeStruct((F, D, B), jnp.float32),
        mesh=_MESH,
        scratch_types=[
            pltpu.VMEM((V,), jnp.float32),
            pltpu.VMEM((B,), jnp.int32),
            pltpu.VMEM((QB,), jnp.float32),
            pltpu.VMEM((QB,), jnp.float32),
            pltpu.SemaphoreType.DMA,
            pltpu.SemaphoreType.DMA,
            pltpu.SemaphoreType.DMA,
            pltpu.SemaphoreType.DMA,
        ],
        compiler_params=pltpu.CompilerParams(
            use_tc_tiling_on_sc=True, needs_layout_passes=False
        ),
    )
    out_t = run(xt, tab_t)
    return out_t.transpose(2, 0, 1)


# speedup vs baseline: 1.0324x; 1.0324x over previous
import jax
import jax.numpy as jnp
from jax import lax
from jax.experimental import pallas as pl
from jax.experimental.pallas import tpu as pltpu
from jax.experimental.pallas import tpu_sc as plsc

F = 26
V = 100000
D = 16
B = 16384

NC = 2
NS = 16
NW = NC * NS
L = 16

UNITS = F * D
UPW = UNITS // NW
QB = B // 4

_MESH = plsc.VectorSubcoreMesh(
    core_axis_name="c", subcore_axis_name="s", num_cores=NC, num_subcores=NS
)


def _body(xt_hbm, tab_hbm, out_hbm, row_v, xcol_v, ob0_v, ob1_v,
          sem_row, sem_row2, sem_w0, sem_w1):
    wid = lax.axis_index("s") * NC + lax.axis_index("c")
    obufs = (ob0_v, ob1_v)
    wsems = (sem_w0, sem_w1)
    pending = [None, None]

    for i in range(UPW):
        u = wid * UPW + i
        f = u // D
        d = u % D

        rowcp = pltpu.async_copy(tab_hbm.at[f, d], row_v, sem_row)
        if i == 0:
            pltpu.sync_copy(xt_hbm.at[f], xcol_v)
        else:
            @pl.when(d == 0)
            def _():
                pltpu.sync_copy(xt_hbm.at[f], xcol_v)
        rowcp.wait()

        for q in range(4):
            s = q % 2
            if pending[s] is not None:
                pending[s].wait()

            @plsc.parallel_loop(0, QB, step=L, unroll=8)
            def gather(k):
                idx = xcol_v[pl.ds(q * QB + k, L)]
                obufs[s][pl.ds(k, L)] = plsc.load_gather(row_v, [idx])

            pending[s] = pltpu.async_copy(
                obufs[s], out_hbm.at[f, d, pl.ds(q * QB, QB)], wsems[s]
            )

    pending[0].wait()
    pending[1].wait()


@jax.jit
def kernel(x, tables):
    xt = x.astype(jnp.int32).T
    tab_t = jnp.swapaxes(tables, 1, 2)
    run = pl.kernel(
        _body,
        out_type=jax.ShapeDtypeStruct((F, D, B), jnp.float32),
        mesh=_MESH,
        scratch_types=[
            pltpu.VMEM((V,), jnp.float32),
            pltpu.VMEM((B,), jnp.int32),
            pltpu.VMEM((QB,), jnp.float32),
            pltpu.VMEM((QB,), jnp.float32),
            pltpu.SemaphoreType.DMA,
            pltpu.SemaphoreType.DMA,
            pltpu.SemaphoreType.DMA,
            pltpu.SemaphoreType.DMA,
        ],
        compiler_params=pltpu.CompilerParams(
            use_tc_tiling_on_sc=True, needs_layout_passes=False
        ),
    )
    out_t = run(xt, tab_t)
    return out_t.transpose(2, 0, 1)
